# baseline (speedup 1.0000x reference)
import jax
import jax.numpy as jnp
from jax.experimental import pallas as pl
from jax.experimental.pallas import tpu as pltpu

N, L, H = 256, 2, 512

_DN = (((1,), (1,)), ((), ()))


def _gru2_kernel(x_ref, h_ref, m_ref,
                 wih0_ref, whh0_ref, wih1_ref, whh1_ref,
                 bi_ref, bh_ref,
                 out_ref, newh_ref,
                 a_s, b_s, mh1_s, r_s, z_s, gi_s, gh_s):
    j = pl.program_id(0)
    gate = jax.lax.rem(j, 3)

    @pl.when(j == 0)
    def _():
        m = jnp.broadcast_to(m_ref[...], (N, H))
        a_s[...] = x_ref[...]
        b_s[...] = h_ref[:, 0, :] * m
        mh1_s[...] = h_ref[:, 1, :] * m

    a = a_s[...]
    b = b_s[...]

    @pl.when(j < 3)
    def _():
        gi_s[...] = jax.lax.dot_general(
            a, wih0_ref[...], _DN, preferred_element_type=jnp.float32)
        gh_s[...] = jax.lax.dot_general(
            b, whh0_ref[...], _DN, preferred_element_type=jnp.float32)

    @pl.when(j >= 3)
    def _():
        gi_s[...] = jax.lax.dot_general(
            a, wih1_ref[...], _DN, preferred_element_type=jnp.float32)
        gh_s[...] = jax.lax.dot_general(
            b, whh1_ref[...], _DN, preferred_element_type=jnp.float32)

    gi = gi_s[...] + bi_ref[0]
    gh = gh_s[...] + bh_ref[0]

    @pl.when(gate == 0)
    def _():
        r_s[...] = jax.nn.sigmoid(gi + gh)

    @pl.when(gate == 1)
    def _():
        z_s[...] = jax.nn.sigmoid(gi + gh)

    @pl.when(gate == 2)
    def _():
        n = jnp.tanh(gi + r_s[...] * gh)
        z = z_s[...]
        hn = (1.0 - z) * n + z * b

        @pl.when(j == 2)
        def _():
            newh_ref[:, 0, :] = hn
            a_s[...] = hn
            b_s[...] = mh1_s[...]

        @pl.when(j == 5)
        def _():
            newh_ref[:, 1, :] = hn
            out_ref[...] = hn


def kernel(x, hidden_states, masks, W_ih0, W_hh0, b_ih0, b_hh0,
           W_ih1, W_hh1, b_ih1, b_hh1):
    m = masks.astype(jnp.float32)
    b_i = jnp.concatenate([b_ih0, b_ih1]).reshape(6, 1, H)
    b_h = jnp.concatenate([b_hh0, b_hh1]).reshape(6, 1, H)

    full = lambda shape: pl.BlockSpec(shape, lambda j: (0,) * len(shape))
    w0_spec = pl.BlockSpec((H, H), lambda j: (jnp.minimum(j, 2), 0))
    w1_spec = pl.BlockSpec((H, H), lambda j: (jnp.maximum(j - 3, 0), 0))
    bias_spec = pl.BlockSpec((1, 1, H), lambda j: (j, 0, 0))
    newh_spec = full((N, L, H))

    out, new_h = pl.pallas_call(
        _gru2_kernel,
        grid=(6,),
        in_specs=[
            full((N, H)),
            full((N, L, H)),
            full((N, 1)),
            w0_spec, w0_spec,
            w1_spec, w1_spec,
            bias_spec, bias_spec,
        ],
        out_specs=(full((N, H)), newh_spec),
        out_shape=(
            jax.ShapeDtypeStruct((N, H), jnp.float32),
            jax.ShapeDtypeStruct((N, L, H), jnp.float32),
        ),
        scratch_shapes=[pltpu.VMEM((N, H), jnp.float32)] * 7,
    )(x, hidden_states, m, W_ih0, W_hh0, W_ih1, W_hh1, b_i, b_h)
    return (out, new_h)


# speedup vs baseline: 1.0014x; 1.0014x over previous
import jax
import jax.numpy as jnp
from jax.experimental import pallas as pl
from jax.experimental.pallas import tpu as pltpu

N, L, H = 256, 2, 512

_DN = (((1,), (1,)), ((), ()))


def _gru2_kernel(x_ref, h_ref, m_ref,
                 wih0_ref, whh0_ref, wih1_ref, whh1_ref,
                 bi_ref, bh_ref,
                 out_ref, newh_ref,
                 a_s, b_s, mh1_s, r_s, z_s, gi_s, gh_s):
    j = pl.program_id(0)
    gate = jax.lax.rem(j, 3)

    @pl.when(j == 0)
    def _():
        m = jnp.broadcast_to(m_ref[...], (N, H))
        a_s[...] = x_ref[...]
        b_s[...] = h_ref[:, 0, :] * m
        mh1_s[...] = h_ref[:, 1, :] * m

    a = a_s[...].astype(jnp.bfloat16)
    b = b_s[...]
    bb = b.astype(jnp.bfloat16)

    @pl.when(j < 3)
    def _():
        gi_s[...] = jax.lax.dot_general(
            a, wih0_ref[...].astype(jnp.bfloat16), _DN,
            preferred_element_type=jnp.float32)
        gh_s[...] = jax.lax.dot_general(
            bb, whh0_ref[...].astype(jnp.bfloat16), _DN,
            preferred_element_type=jnp.float32)

    @pl.when(j >= 3)
    def _():
        gi_s[...] = jax.lax.dot_general(
            a, wih1_ref[...].astype(jnp.bfloat16), _DN,
            preferred_element_type=jnp.float32)
        gh_s[...] = jax.lax.dot_general(
            bb, whh1_ref[...].astype(jnp.bfloat16), _DN,
            preferred_element_type=jnp.float32)

    gi = gi_s[...] + bi_ref[0]
    gh = gh_s[...] + bh_ref[0]

    @pl.when(gate == 0)
    def _():
        r_s[...] = jax.nn.sigmoid(gi + gh)

    @pl.when(gate == 1)
    def _():
        z_s[...] = jax.nn.sigmoid(gi + gh)

    @pl.when(gate == 2)
    def _():
        n = jnp.tanh(gi + r_s[...] * gh)
        z = z_s[...]
        hn = (1.0 - z) * n + z * b

        @pl.when(j == 2)
        def _():
            newh_ref[:, 0, :] = hn
            a_s[...] = hn
            b_s[...] = mh1_s[...]

        @pl.when(j == 5)
        def _():
            newh_ref[:, 1, :] = hn
            out_ref[...] = hn


def kernel(x, hidden_states, masks, W_ih0, W_hh0, b_ih0, b_hh0,
           W_ih1, W_hh1, b_ih1, b_hh1):
    m = masks.astype(jnp.float32)
    b_i = jnp.concatenate([b_ih0, b_ih1]).reshape(6, 1, H)
    b_h = jnp.concatenate([b_hh0, b_hh1]).reshape(6, 1, H)

    full = lambda shape: pl.BlockSpec(shape, lambda j: (0,) * len(shape))
    w0_spec = pl.BlockSpec((H, H), lambda j: (jnp.minimum(j, 2), 0))
    w1_spec = pl.BlockSpec((H, H), lambda j: (jnp.maximum(j - 3, 0), 0))
    bias_spec = pl.BlockSpec((1, 1, H), lambda j: (j, 0, 0))
    newh_spec = full((N, L, H))

    out, new_h = pl.pallas_call(
        _gru2_kernel,
        grid=(6,),
        in_specs=[
            full((N, H)),
            full((N, L, H)),
            full((N, 1)),
            w0_spec, w0_spec,
            w1_spec, w1_spec,
            bias_spec, bias_spec,
        ],
        out_specs=(full((N, H)), newh_spec),
        out_shape=(
            jax.ShapeDtypeStruct((N, H), jnp.float32),
            jax.ShapeDtypeStruct((N, L, H), jnp.float32),
        ),
        scratch_shapes=[pltpu.VMEM((N, H), jnp.float32)] * 7,
    )(x, hidden_states, m, W_ih0, W_hh0, W_ih1, W_hh1, b_i, b_h)
    return (out, new_h)
